# Initial kernel scaffold; baseline (speedup 1.0000x reference)
import jax
import jax.numpy as jnp
from jax.experimental import pallas as pl


def kernel(x, Wg, W1, b1, W2, b2):
    raise NotImplementedError("write your pallas kernel here")


# speedup vs baseline: 2.2051x; 2.2051x over previous
import functools
import math

import jax
import jax.numpy as jnp
from jax.experimental import pallas as pl
from jax.experimental.pallas import tpu as pltpu

D_MODEL_ = 1024
D_FF_ = 4096
NE_ = 8
TK_ = 2
T_TILE = 256


F_BLK = 2048
NF_ = D_FF_ // F_BLK


def _moe_body(x_ref, wg_ref, w1_ref, b1_ref, w2_ref, b2_ref, out_ref, acc_ref):
    e = pl.program_id(0)
    f = pl.program_id(1)
    t = pl.program_id(2)

    x = x_ref[...]

    logits = jnp.dot(x, wg_ref[...], preferred_element_type=jnp.float32)
    col = jax.lax.broadcasted_iota(jnp.int32, logits.shape, 1)
    m1 = jnp.max(logits, axis=1, keepdims=True)
    a1 = jnp.min(jnp.where(logits == m1, col, NE_), axis=1, keepdims=True)
    masked = jnp.where(col == a1, -jnp.inf, logits)
    m2 = jnp.max(masked, axis=1, keepdims=True)
    a2 = jnp.min(jnp.where(masked == m2, col, NE_), axis=1, keepdims=True)
    d = jnp.exp(m2 - m1)
    p1 = 1.0 / (1.0 + d)
    p2 = d / (1.0 + d)
    w = jnp.where(a1 == e, p1, 0.0) + jnp.where(a2 == e, p2, 0.0)

    h = jnp.dot(x, w1_ref[0], preferred_element_type=jnp.float32) + b1_ref[0]
    h = 0.5 * h * (1.0 + jax.lax.erf(h / math.sqrt(2.0)))
    y = jnp.dot(h, w2_ref[0], preferred_element_type=jnp.float32)
    contrib = w * y
    contrib += jnp.where(f == 0, 1.0, 0.0) * (w * b2_ref[0])

    @pl.when((e == 0) & (f == 0))
    def _():
        acc_ref[pl.ds(t * T_TILE, T_TILE), :] = contrib

    @pl.when((e > 0) | (f > 0))
    def _():
        acc_ref[pl.ds(t * T_TILE, T_TILE), :] += contrib

    @pl.when((e == NE_ - 1) & (f == NF_ - 1))
    def _():
        out_ref[...] = acc_ref[pl.ds(t * T_TILE, T_TILE), :]


def kernel(x, Wg, W1, b1, W2, b2):
    B, S, d = x.shape
    xf = x.reshape(-1, d)
    T = xf.shape[0]
    n_t = T // T_TILE

    out = pl.pallas_call(
        _moe_body,
        grid=(NE_, NF_, n_t),
        in_specs=[
            pl.BlockSpec((T_TILE, d), lambda e, f, t: (t, 0)),
            pl.BlockSpec((d, NE_), lambda e, f, t: (0, 0)),
            pl.BlockSpec((1, d, F_BLK), lambda e, f, t: (e, 0, f)),
            pl.BlockSpec((1, 1, F_BLK), lambda e, f, t: (e, 0, f)),
            pl.BlockSpec((1, F_BLK, d), lambda e, f, t: (e, f, 0)),
            pl.BlockSpec((1, 1, d), lambda e, f, t: (e, 0, 0)),
        ],
        out_specs=pl.BlockSpec((T_TILE, d), lambda e, f, t: (t, 0)),
        out_shape=jax.ShapeDtypeStruct((T, d), jnp.float32),
        scratch_shapes=[pltpu.VMEM((T, d), jnp.float32)],
        compiler_params=pltpu.CompilerParams(
            dimension_semantics=("arbitrary", "arbitrary", "arbitrary"),
        ),
    )(xf, Wg, W1, b1.reshape(NE_, 1, D_FF_), W2, b2.reshape(NE_, 1, d))
    return out.reshape(B, S, d)
